# baseline (device time: 185007 ns/iter reference)
import jax
import jax.numpy as jnp
from jax import lax
from jax.experimental import pallas as pl
from jax.experimental.pallas import tpu as pltpu

N_DEV = 4


def kernel(x, w_mat):
    m, _ = x.shape
    _, n = w_mat.shape
    ch = m // N_DEV
    hh = ch // 2
    sg = 256

    def body(x_ref, w_ref, out_ref, ybuf, sbuf_r, sbuf_l, pbuf_r, pbuf_l,
             am_ref,
             rs_send_r, rs_recv_r, rs_send_l, rs_recv_l,
             ag_send_r, ag_recv_r, ag_send_l, ag_recv_l,
             am_send, am_recv, st_sems):
        my = lax.axis_index("i")
        left = (my + N_DEV - 1) % N_DEV
        right = (my + 1) % N_DEV

        def rcopy(src, dst, ss, rs, dev):
            return pltpu.make_async_remote_copy(
                src_ref=src, dst_ref=dst, send_sem=ss, recv_sem=rs,
                device_id=(dev,), device_id_type=pl.DeviceIdType.MESH)

        barrier = pltpu.get_barrier_semaphore()
        for nbr in (left, right):
            pl.semaphore_signal(barrier, inc=1, device_id=(nbr,),
                                device_id_type=pl.DeviceIdType.MESH)
        pl.semaphore_wait(barrier, 2)

        def rrow(c, s):
            return pl.ds(c * ch + s * sg, sg)

        def lrow(c, s):
            return pl.ds(c * ch + hh + s * sg, sg)

        def sseg(s):
            return pl.ds(s * sg, sg)

        def dot_seg(rows, s, dst):
            part = jnp.dot(
                x_ref[rows, :],
                w_ref[:, :], preferred_element_type=jnp.float32)
            dst[sseg(s), :] = part.astype(jnp.bfloat16)

        st_ops = []

        def store_rows(rows):
            op = pltpu.make_async_copy(
                ybuf.at[rows], out_ref.at[rows], st_sems.at[len(st_ops)])
            op.start()
            st_ops.append(op)

        cr0 = (my + N_DEV - 1) % N_DEV
        cl0 = (my + 1) % N_DEV
        for s in range(2):
            dot_seg(rrow(cr0, s), s, sbuf_r)
            dot_seg(lrow(cl0, s), s, sbuf_l)
        send_r = [None, None]
        send_l = [None, None]
        for s in range(2):
            send_r[s] = rcopy(sbuf_r.at[sseg(s)], ybuf.at[rrow(cr0, s)],
                              rs_send_r.at[0, s], rs_recv_r.at[0, s], right)
            send_l[s] = rcopy(sbuf_l.at[sseg(s)], ybuf.at[lrow(cl0, s)],
                              rs_send_l.at[0, s], rs_recv_l.at[0, s], left)
            send_r[s].start()
            send_l[s].start()

        amax_mine = jnp.float32(0.0)
        for h in range(N_DEV - 1):
            nr = (my + 2 * N_DEV - h - 2) % N_DEV if h < N_DEV - 2 else my
            nl = (my + h + 2) % N_DEV if h < N_DEV - 2 else my
            for s in range(2):
                dot_seg(rrow(nr, s), s, pbuf_r)
                dot_seg(lrow(nl, s), s, pbuf_l)
            for s in range(2):
                rcopy(sbuf_r.at[sseg(s)], ybuf.at[rrow(nr, s)],
                      rs_send_r.at[h, s], rs_recv_r.at[h, s],
                      right).wait_recv()
                rcopy(sbuf_l.at[sseg(s)], ybuf.at[lrow(nl, s)],
                      rs_send_l.at[h, s], rs_recv_l.at[h, s],
                      left).wait_recv()
                if h < N_DEV - 2:
                    send_r[s].wait_send()
                    send_l[s].wait_send()
                    acc_r = (pbuf_r[sseg(s), :].astype(jnp.float32)
                             + ybuf[rrow(nr, s), :].astype(jnp.float32))
                    sbuf_r[sseg(s), :] = acc_r.astype(jnp.bfloat16)
                    acc_l = (pbuf_l[sseg(s), :].astype(jnp.float32)
                             + ybuf[lrow(nl, s), :].astype(jnp.float32))
                    sbuf_l[sseg(s), :] = acc_l.astype(jnp.bfloat16)
                    send_r[s] = rcopy(
                        sbuf_r.at[sseg(s)], ybuf.at[rrow(nr, s)],
                        rs_send_r.at[h + 1, s], rs_recv_r.at[h + 1, s],
                        right)
                    send_l[s] = rcopy(
                        sbuf_l.at[sseg(s)], ybuf.at[lrow(nl, s)],
                        rs_send_l.at[h + 1, s], rs_recv_l.at[h + 1, s],
                        left)
                    send_r[s].start()
                    send_l[s].start()
                else:
                    acc_r = (pbuf_r[sseg(s), :].astype(jnp.float32)
                             + ybuf[rrow(my, s), :].astype(jnp.float32))
                    y_r = acc_r.astype(jnp.bfloat16)
                    ybuf[rrow(my, s), :] = y_r
                    amax_mine = jnp.maximum(
                        amax_mine,
                        jnp.max(jnp.abs(y_r.astype(jnp.float32))))
                    acc_l = (pbuf_l[sseg(s), :].astype(jnp.float32)
                             + ybuf[lrow(my, s), :].astype(jnp.float32))
                    y_l = acc_l.astype(jnp.bfloat16)
                    ybuf[lrow(my, s), :] = y_l
                    amax_mine = jnp.maximum(
                        amax_mine,
                        jnp.max(jnp.abs(y_l.astype(jnp.float32))))
        send_r[0].wait_send()
        send_l[0].wait_send()
        send_r[1].wait_send()
        send_l[1].wait_send()

        am_ref[pl.ds(my, 1), :, :] = jnp.full((1, 8, 128), amax_mine,
                                              jnp.float32)
        am_ops = []
        for k in range(1, N_DEV):
            op = rcopy(am_ref.at[my], am_ref.at[my],
                       am_send.at[k - 1], am_recv.at[k - 1],
                       (my + k) % N_DEV)
            op.start()
            am_ops.append(op)
        for op in am_ops:
            op.wait()

        amax = jnp.max(am_ref[:, :, :])
        scale = amax / 127.0

        for b in range(ch // sg):
            rows = pl.ds(my * ch + b * sg, sg)
            y = ybuf[rows, :].astype(jnp.float32)
            q = jnp.clip(jnp.round(y / scale), -127.0, 127.0)
            ybuf[rows, :] = (q * scale).astype(jnp.bfloat16)
            store_rows(rows)

        ag_ops = []

        def ag_start(h, s):
            sr = (my + N_DEV - h) % N_DEV
            sl = (my + h) % N_DEV
            opr = rcopy(ybuf.at[rrow(sr, s)], ybuf.at[rrow(sr, s)],
                        ag_send_r.at[h, s], ag_recv_r.at[h, s], right)
            opl = rcopy(ybuf.at[lrow(sl, s)], ybuf.at[lrow(sl, s)],
                        ag_send_l.at[h, s], ag_recv_l.at[h, s], left)
            opr.start()
            opl.start()
            ag_ops.extend([opr, opl])

        def ag_wait_recv(h, s):
            rr = (my + 2 * N_DEV - h - 1) % N_DEV
            rl = (my + h + 1) % N_DEV
            rcopy(ybuf.at[rrow(rr, s)], ybuf.at[rrow(rr, s)],
                  ag_send_r.at[h, s], ag_recv_r.at[h, s], right).wait_recv()
            rcopy(ybuf.at[lrow(rl, s)], ybuf.at[lrow(rl, s)],
                  ag_send_l.at[h, s], ag_recv_l.at[h, s], left).wait_recv()
            store_rows(rrow(rr, s))
            store_rows(lrow(rl, s))

        ag_start(0, 0)
        ag_start(0, 1)
        for h in range(1, N_DEV - 1):
            for s in range(2):
                ag_wait_recv(h - 1, s)
                ag_start(h, s)
        for s in range(2):
            ag_wait_recv(N_DEV - 2, s)
        for op in ag_ops:
            op.wait_send()
        for op in st_ops:
            op.wait()

    dma3 = pltpu.SemaphoreType.DMA((N_DEV - 1,))
    dma32 = pltpu.SemaphoreType.DMA((N_DEV - 1, 2))
    return pl.pallas_call(
        body,
        out_shape=jax.ShapeDtypeStruct((m, n), jnp.bfloat16),
        in_specs=[pl.BlockSpec(memory_space=pltpu.VMEM),
                  pl.BlockSpec(memory_space=pltpu.VMEM)],
        out_specs=pl.BlockSpec(memory_space=pltpu.MemorySpace.HBM),
        scratch_shapes=[
            pltpu.VMEM((m, n), jnp.bfloat16),
            pltpu.VMEM((hh, n), jnp.bfloat16),
            pltpu.VMEM((hh, n), jnp.bfloat16),
            pltpu.VMEM((hh, n), jnp.bfloat16),
            pltpu.VMEM((hh, n), jnp.bfloat16),
            pltpu.VMEM((N_DEV, 8, 128), jnp.float32),
            dma32, dma32,
            dma32, dma32,
            dma32, dma32,
            dma32, dma32,
            dma3, dma3,
            pltpu.SemaphoreType.DMA((16,)),
        ],
        compiler_params=pltpu.CompilerParams(
            collective_id=0, vmem_limit_bytes=44 * 1024 * 1024),
    )(x.astype(jnp.bfloat16), w_mat.astype(jnp.bfloat16))


# device time: 178026 ns/iter; 1.0392x vs baseline; 1.0392x over previous
import jax
import jax.numpy as jnp
from jax import lax
from jax.experimental import pallas as pl
from jax.experimental.pallas import tpu as pltpu

N_DEV = 4


def kernel(x, w_mat):
    m, _ = x.shape
    _, n = w_mat.shape
    ch = m // N_DEV
    hh = ch // 2
    sg = 256

    def body(x_ref, w_ref, out_ref, sbuf_r, sbuf_l, pbuf_r, pbuf_l, am_ref,
             rs_send_r, rs_recv_r, rs_send_l, rs_recv_l,
             ag_send_r, ag_recv_r, ag_send_l, ag_recv_l,
             am_send, am_recv):
        my = lax.axis_index("i")
        left = (my + N_DEV - 1) % N_DEV
        right = (my + 1) % N_DEV

        def rcopy(src, dst, ss, rs, dev):
            return pltpu.make_async_remote_copy(
                src_ref=src, dst_ref=dst, send_sem=ss, recv_sem=rs,
                device_id=(dev,), device_id_type=pl.DeviceIdType.MESH)

        barrier = pltpu.get_barrier_semaphore()
        for nbr in (left, right):
            pl.semaphore_signal(barrier, inc=1, device_id=(nbr,),
                                device_id_type=pl.DeviceIdType.MESH)

        def rrow(c, s):
            return pl.ds(c * ch + s * sg, sg)

        def lrow(c, s):
            return pl.ds(c * ch + hh + s * sg, sg)

        def sseg(s):
            return pl.ds(s * sg, sg)

        def dot_seg(rows, s, dst):
            part = jnp.dot(
                x_ref[rows, :],
                w_ref[:, :], preferred_element_type=jnp.float32)
            dst[sseg(s), :] = part.astype(jnp.bfloat16)

        cr0 = (my + N_DEV - 1) % N_DEV
        cl0 = (my + 1) % N_DEV
        send_r = [None, None]
        send_l = [None, None]
        dot_seg(rrow(cr0, 0), 0, sbuf_r)
        pl.semaphore_wait(barrier, 2)
        send_r[0] = rcopy(sbuf_r.at[sseg(0)], out_ref.at[rrow(cr0, 0)],
                          rs_send_r.at[0, 0], rs_recv_r.at[0, 0], right)
        send_r[0].start()
        dot_seg(lrow(cl0, 0), 0, sbuf_l)
        send_l[0] = rcopy(sbuf_l.at[sseg(0)], out_ref.at[lrow(cl0, 0)],
                          rs_send_l.at[0, 0], rs_recv_l.at[0, 0], left)
        send_l[0].start()
        dot_seg(rrow(cr0, 1), 1, sbuf_r)
        send_r[1] = rcopy(sbuf_r.at[sseg(1)], out_ref.at[rrow(cr0, 1)],
                          rs_send_r.at[0, 1], rs_recv_r.at[0, 1], right)
        send_r[1].start()
        dot_seg(lrow(cl0, 1), 1, sbuf_l)
        send_l[1] = rcopy(sbuf_l.at[sseg(1)], out_ref.at[lrow(cl0, 1)],
                          rs_send_l.at[0, 1], rs_recv_l.at[0, 1], left)
        send_l[1].start()

        amax_mine = jnp.float32(0.0)
        for h in range(N_DEV - 1):
            nr = (my + 2 * N_DEV - h - 2) % N_DEV if h < N_DEV - 2 else my
            nl = (my + h + 2) % N_DEV if h < N_DEV - 2 else my
            for s in range(2):
                dot_seg(rrow(nr, s), s, pbuf_r)
                dot_seg(lrow(nl, s), s, pbuf_l)
            for s in range(2):
                rcopy(sbuf_r.at[sseg(s)], out_ref.at[rrow(nr, s)],
                      rs_send_r.at[h, s], rs_recv_r.at[h, s],
                      right).wait_recv()
                rcopy(sbuf_l.at[sseg(s)], out_ref.at[lrow(nl, s)],
                      rs_send_l.at[h, s], rs_recv_l.at[h, s],
                      left).wait_recv()
                if h < N_DEV - 2:
                    send_r[s].wait_send()
                    send_l[s].wait_send()
                    acc_r = (pbuf_r[sseg(s), :].astype(jnp.float32)
                             + out_ref[rrow(nr, s), :].astype(jnp.float32))
                    sbuf_r[sseg(s), :] = acc_r.astype(jnp.bfloat16)
                    acc_l = (pbuf_l[sseg(s), :].astype(jnp.float32)
                             + out_ref[lrow(nl, s), :].astype(jnp.float32))
                    sbuf_l[sseg(s), :] = acc_l.astype(jnp.bfloat16)
                    send_r[s] = rcopy(
                        sbuf_r.at[sseg(s)], out_ref.at[rrow(nr, s)],
                        rs_send_r.at[h + 1, s], rs_recv_r.at[h + 1, s],
                        right)
                    send_l[s] = rcopy(
                        sbuf_l.at[sseg(s)], out_ref.at[lrow(nl, s)],
                        rs_send_l.at[h + 1, s], rs_recv_l.at[h + 1, s],
                        left)
                    send_r[s].start()
                    send_l[s].start()
                else:
                    acc_r = (pbuf_r[sseg(s), :].astype(jnp.float32)
                             + out_ref[rrow(my, s), :].astype(jnp.float32))
                    y_r = acc_r.astype(jnp.bfloat16)
                    out_ref[rrow(my, s), :] = y_r
                    amax_mine = jnp.maximum(
                        amax_mine,
                        jnp.max(jnp.abs(y_r.astype(jnp.float32))))
                    acc_l = (pbuf_l[sseg(s), :].astype(jnp.float32)
                             + out_ref[lrow(my, s), :].astype(jnp.float32))
                    y_l = acc_l.astype(jnp.bfloat16)
                    out_ref[lrow(my, s), :] = y_l
                    amax_mine = jnp.maximum(
                        amax_mine,
                        jnp.max(jnp.abs(y_l.astype(jnp.float32))))
        send_r[0].wait_send()
        send_l[0].wait_send()
        send_r[1].wait_send()
        send_l[1].wait_send()

        am_ref[pl.ds(my, 1), :, :] = jnp.full((1, 8, 128), amax_mine,
                                              jnp.float32)
        am_ops = []
        for k in range(1, N_DEV):
            op = rcopy(am_ref.at[my], am_ref.at[my],
                       am_send.at[k - 1], am_recv.at[k - 1],
                       (my + k) % N_DEV)
            op.start()
            am_ops.append(op)
        for op in am_ops:
            op.wait()

        amax = jnp.max(am_ref[:, :, :])
        scale = amax / 127.0

        ag_ops = []

        def ag_start_r(h, s):
            sr = (my + N_DEV - h) % N_DEV
            opr = rcopy(out_ref.at[rrow(sr, s)], out_ref.at[rrow(sr, s)],
                        ag_send_r.at[h, s], ag_recv_r.at[h, s], right)
            opr.start()
            ag_ops.append(opr)

        def ag_start_l(h, s):
            sl = (my + h) % N_DEV
            opl = rcopy(out_ref.at[lrow(sl, s)], out_ref.at[lrow(sl, s)],
                        ag_send_l.at[h, s], ag_recv_l.at[h, s], left)
            opl.start()
            ag_ops.append(opl)

        def ag_wait_recv(h, s):
            rr = (my + 2 * N_DEV - h - 1) % N_DEV
            rl = (my + h + 1) % N_DEV
            rcopy(out_ref.at[rrow(rr, s)], out_ref.at[rrow(rr, s)],
                  ag_send_r.at[h, s], ag_recv_r.at[h, s], right).wait_recv()
            rcopy(out_ref.at[lrow(rl, s)], out_ref.at[lrow(rl, s)],
                  ag_send_l.at[h, s], ag_recv_l.at[h, s], left).wait_recv()

        def quant_rows(rows):
            y = out_ref[rows, :].astype(jnp.float32)
            q = jnp.clip(jnp.round(y / scale), -127.0, 127.0)
            out_ref[rows, :] = (q * scale).astype(jnp.bfloat16)

        quant_rows(rrow(my, 0))
        ag_start_r(0, 0)
        quant_rows(lrow(my, 0))
        ag_start_l(0, 0)
        quant_rows(rrow(my, 1))
        ag_start_r(0, 1)
        quant_rows(lrow(my, 1))
        ag_start_l(0, 1)
        for h in range(1, N_DEV - 1):
            for s in range(2):
                ag_wait_recv(h - 1, s)
                ag_start_r(h, s)
                ag_start_l(h, s)
        for s in range(2):
            ag_wait_recv(N_DEV - 2, s)
        for op in ag_ops:
            op.wait_send()

    dma3 = pltpu.SemaphoreType.DMA((N_DEV - 1,))
    dma32 = pltpu.SemaphoreType.DMA((N_DEV - 1, 2))
    return pl.pallas_call(
        body,
        out_shape=jax.ShapeDtypeStruct((m, n), jnp.bfloat16),
        in_specs=[pl.BlockSpec(memory_space=pltpu.VMEM),
                  pl.BlockSpec(memory_space=pltpu.VMEM)],
        out_specs=pl.BlockSpec(memory_space=pltpu.VMEM),
        scratch_shapes=[
            pltpu.VMEM((hh, n), jnp.bfloat16),
            pltpu.VMEM((hh, n), jnp.bfloat16),
            pltpu.VMEM((hh, n), jnp.bfloat16),
            pltpu.VMEM((hh, n), jnp.bfloat16),
            pltpu.VMEM((N_DEV, 8, 128), jnp.float32),
            dma32, dma32,
            dma32, dma32,
            dma32, dma32,
            dma32, dma32,
            dma3, dma3,
        ],
        compiler_params=pltpu.CompilerParams(
            collective_id=0, vmem_limit_bytes=36 * 1024 * 1024),
    )(x.astype(jnp.bfloat16), w_mat.astype(jnp.bfloat16))


# device time: 144948 ns/iter; 1.2764x vs baseline; 1.2282x over previous
import jax
import jax.numpy as jnp
from jax import lax
from jax.experimental import pallas as pl
from jax.experimental.pallas import tpu as pltpu

N_DEV = 4


def kernel(x, w_mat):
    m, _ = x.shape
    _, n = w_mat.shape
    ch = m // N_DEV
    hh = ch // 2
    sg = 256

    def body(x_ref, w_ref, out_ref, sbuf_r, sbuf_l, pbuf_r, pbuf_l, qbuf,
             am_ref,
             rs_send_r, rs_recv_r, rs_send_l, rs_recv_l,
             ag_send_r, ag_recv_r, ag_send_l, ag_recv_l,
             am_send, am_recv):
        my = lax.axis_index("i")
        left = (my + N_DEV - 1) % N_DEV
        right = (my + 1) % N_DEV

        def rcopy(src, dst, ss, rs, dev):
            return pltpu.make_async_remote_copy(
                src_ref=src, dst_ref=dst, send_sem=ss, recv_sem=rs,
                device_id=(dev,), device_id_type=pl.DeviceIdType.MESH)

        barrier = pltpu.get_barrier_semaphore()
        for nbr in (left, right):
            pl.semaphore_signal(barrier, inc=1, device_id=(nbr,),
                                device_id_type=pl.DeviceIdType.MESH)

        def rrow(c, s):
            return pl.ds(c * ch + s * sg, sg)

        def lrow(c, s):
            return pl.ds(c * ch + hh + s * sg, sg)

        def sseg(s):
            return pl.ds(s * sg, sg)

        def dot_seg(rows, s, dst):
            part = jnp.dot(
                x_ref[rows, :],
                w_ref[:, :], preferred_element_type=jnp.float32)
            dst[sseg(s), :] = part.astype(jnp.bfloat16)

        cr0 = (my + N_DEV - 1) % N_DEV
        cl0 = (my + 1) % N_DEV
        send_r = [None, None]
        send_l = [None, None]
        dot_seg(rrow(cr0, 0), 0, sbuf_r)
        pl.semaphore_wait(barrier, 2)
        send_r[0] = rcopy(sbuf_r.at[sseg(0)], out_ref.at[rrow(cr0, 0)],
                          rs_send_r.at[0, 0], rs_recv_r.at[0, 0], right)
        send_r[0].start()
        dot_seg(lrow(cl0, 0), 0, sbuf_l)
        send_l[0] = rcopy(sbuf_l.at[sseg(0)], out_ref.at[lrow(cl0, 0)],
                          rs_send_l.at[0, 0], rs_recv_l.at[0, 0], left)
        send_l[0].start()
        dot_seg(rrow(cr0, 1), 1, sbuf_r)
        send_r[1] = rcopy(sbuf_r.at[sseg(1)], out_ref.at[rrow(cr0, 1)],
                          rs_send_r.at[0, 1], rs_recv_r.at[0, 1], right)
        send_r[1].start()
        dot_seg(lrow(cl0, 1), 1, sbuf_l)
        send_l[1] = rcopy(sbuf_l.at[sseg(1)], out_ref.at[lrow(cl0, 1)],
                          rs_send_l.at[0, 1], rs_recv_l.at[0, 1], left)
        send_l[1].start()

        amax_mine = jnp.float32(0.0)
        for h in range(N_DEV - 1):
            nr = (my + 2 * N_DEV - h - 2) % N_DEV if h < N_DEV - 2 else my
            nl = (my + h + 2) % N_DEV if h < N_DEV - 2 else my
            for s in range(2):
                dot_seg(rrow(nr, s), s, pbuf_r)
                dot_seg(lrow(nl, s), s, pbuf_l)
            for s in range(2):
                rcopy(sbuf_r.at[sseg(s)], out_ref.at[rrow(nr, s)],
                      rs_send_r.at[h, s], rs_recv_r.at[h, s],
                      right).wait_recv()
                rcopy(sbuf_l.at[sseg(s)], out_ref.at[lrow(nl, s)],
                      rs_send_l.at[h, s], rs_recv_l.at[h, s],
                      left).wait_recv()
                if h < N_DEV - 2:
                    send_r[s].wait_send()
                    send_l[s].wait_send()
                    acc_r = (pbuf_r[sseg(s), :].astype(jnp.float32)
                             + out_ref[rrow(nr, s), :].astype(jnp.float32))
                    sbuf_r[sseg(s), :] = acc_r.astype(jnp.bfloat16)
                    acc_l = (pbuf_l[sseg(s), :].astype(jnp.float32)
                             + out_ref[lrow(nl, s), :].astype(jnp.float32))
                    sbuf_l[sseg(s), :] = acc_l.astype(jnp.bfloat16)
                    send_r[s] = rcopy(
                        sbuf_r.at[sseg(s)], out_ref.at[rrow(nr, s)],
                        rs_send_r.at[h + 1, s], rs_recv_r.at[h + 1, s],
                        right)
                    send_l[s] = rcopy(
                        sbuf_l.at[sseg(s)], out_ref.at[lrow(nl, s)],
                        rs_send_l.at[h + 1, s], rs_recv_l.at[h + 1, s],
                        left)
                    send_r[s].start()
                    send_l[s].start()
                else:
                    acc_r = (pbuf_r[sseg(s), :].astype(jnp.float32)
                             + out_ref[rrow(my, s), :].astype(jnp.float32))
                    y_r = acc_r.astype(jnp.bfloat16)
                    out_ref[rrow(my, s), :] = y_r
                    amax_mine = jnp.maximum(
                        amax_mine,
                        jnp.max(jnp.abs(y_r.astype(jnp.float32))))
                    acc_l = (pbuf_l[sseg(s), :].astype(jnp.float32)
                             + out_ref[lrow(my, s), :].astype(jnp.float32))
                    y_l = acc_l.astype(jnp.bfloat16)
                    out_ref[lrow(my, s), :] = y_l
                    amax_mine = jnp.maximum(
                        amax_mine,
                        jnp.max(jnp.abs(y_l.astype(jnp.float32))))
        send_r[0].wait_send()
        send_l[0].wait_send()
        send_r[1].wait_send()
        send_l[1].wait_send()

        am_ref[pl.ds(my, 1), :, :] = jnp.full((1, 8, 128), amax_mine,
                                              jnp.float32)
        am_ops = []
        for k in range(1, N_DEV):
            op = rcopy(am_ref.at[my], am_ref.at[my],
                       am_send.at[k - 1], am_recv.at[k - 1],
                       (my + k) % N_DEV)
            op.start()
            am_ops.append(op)
        for op in am_ops:
            op.wait()

        amax = jnp.max(am_ref[:, :, :])
        scale = amax / 127.0

        ag_ops = []

        def ag_start_r(h, s):
            sr = (my + N_DEV - h) % N_DEV
            opr = rcopy(qbuf.at[rrow(sr, s)], qbuf.at[rrow(sr, s)],
                        ag_send_r.at[h, s], ag_recv_r.at[h, s], right)
            opr.start()
            ag_ops.append(opr)

        def ag_start_l(h, s):
            sl = (my + h) % N_DEV
            opl = rcopy(qbuf.at[lrow(sl, s)], qbuf.at[lrow(sl, s)],
                        ag_send_l.at[h, s], ag_recv_l.at[h, s], left)
            opl.start()
            ag_ops.append(opl)

        def ag_wait_recv(h, s):
            rr = (my + 2 * N_DEV - h - 1) % N_DEV
            rl = (my + h + 1) % N_DEV
            rcopy(qbuf.at[rrow(rr, s)], qbuf.at[rrow(rr, s)],
                  ag_send_r.at[h, s], ag_recv_r.at[h, s], right).wait_recv()
            rcopy(qbuf.at[lrow(rl, s)], qbuf.at[lrow(rl, s)],
                  ag_send_l.at[h, s], ag_recv_l.at[h, s], left).wait_recv()
            return rr, rl

        def quant_rows(rows):
            y = out_ref[rows, :].astype(jnp.float32)
            q = jnp.clip(jnp.round(y / scale), -127.0, 127.0)
            qbuf[rows, :] = q.astype(jnp.int8)
            out_ref[rows, :] = (q * scale).astype(jnp.bfloat16)

        def deq_rows(rows):
            q = qbuf[rows, :].astype(jnp.float32)
            out_ref[rows, :] = (q * scale).astype(jnp.bfloat16)

        quant_rows(rrow(my, 0))
        ag_start_r(0, 0)
        quant_rows(lrow(my, 0))
        ag_start_l(0, 0)
        quant_rows(rrow(my, 1))
        ag_start_r(0, 1)
        quant_rows(lrow(my, 1))
        ag_start_l(0, 1)
        for h in range(1, N_DEV - 1):
            for s in range(2):
                rr, rl = ag_wait_recv(h - 1, s)
                ag_start_r(h, s)
                ag_start_l(h, s)
                deq_rows(rrow(rr, s))
                deq_rows(lrow(rl, s))
        for s in range(2):
            rr, rl = ag_wait_recv(N_DEV - 2, s)
            deq_rows(rrow(rr, s))
            deq_rows(lrow(rl, s))
        for op in ag_ops:
            op.wait_send()

    dma3 = pltpu.SemaphoreType.DMA((N_DEV - 1,))
    dma32 = pltpu.SemaphoreType.DMA((N_DEV - 1, 2))
    return pl.pallas_call(
        body,
        out_shape=jax.ShapeDtypeStruct((m, n), jnp.bfloat16),
        in_specs=[pl.BlockSpec(memory_space=pltpu.VMEM),
                  pl.BlockSpec(memory_space=pltpu.VMEM)],
        out_specs=pl.BlockSpec(memory_space=pltpu.VMEM),
        scratch_shapes=[
            pltpu.VMEM((hh, n), jnp.bfloat16),
            pltpu.VMEM((hh, n), jnp.bfloat16),
            pltpu.VMEM((hh, n), jnp.bfloat16),
            pltpu.VMEM((hh, n), jnp.bfloat16),
            pltpu.VMEM((m, n), jnp.int8),
            pltpu.VMEM((N_DEV, 8, 128), jnp.float32),
            dma32, dma32,
            dma32, dma32,
            dma32, dma32,
            dma32, dma32,
            dma3, dma3,
        ],
        compiler_params=pltpu.CompilerParams(
            collective_id=0, vmem_limit_bytes=36 * 1024 * 1024),
    )(x.astype(jnp.bfloat16), w_mat.astype(jnp.bfloat16))
